# baseline (speedup 1.0000x reference)
import functools

import jax
import jax.numpy as jnp
from jax import lax
from jax.experimental import pallas as pl
from jax.experimental.pallas import tpu as pltpu
from jax.experimental.pallas import tpu_sc as plsc

NC = 2
NS = 16
NW = NC * NS
CHUNK = 128


def _make_gather(n_tri, n_stage, in_ch, ep):
    b_per_w = ep // NW
    n_ch = b_per_w // CHUNK
    rows_w = n_stage // NS
    mesh = plsc.VectorSubcoreMesh(core_axis_name="c", subcore_axis_name="s",
                                  num_cores=NC, num_subcores=NS)

    @functools.partial(
        pl.kernel, mesh=mesh,
        out_type=jax.ShapeDtypeStruct((ep, in_ch), jnp.float32),
        scratch_types=[
            pltpu.VMEM((n_ch, CHUNK), jnp.int32),
            pltpu.VMEM((b_per_w, in_ch), jnp.float32),
            pltpu.SemaphoreType.DMA,
        ],
        compiler_params=pltpu.CompilerParams(use_tc_tiling_on_sc=False),
    )
    def gather(tf_hbm, idx_hbm, f_hbm, idx_v, rows_v, sem):
        cid = lax.axis_index("c")
        sid = lax.axis_index("s")
        wid = sid * NC + cid
        base = wid * b_per_w
        pltpu.sync_copy(idx_hbm.at[wid], idx_v)

        def fire(j, carry):
            pltpu.async_copy(tf_hbm.at[idx_v.at[j]],
                             rows_v.at[pl.ds(j * CHUNK, CHUNK)], sem)
            return carry

        lax.fori_loop(0, n_ch, fire, 0)
        pltpu.make_async_copy(tf_hbm.at[pl.ds(0, b_per_w)], rows_v, sem).wait()
        pltpu.sync_copy(rows_v, f_hbm.at[pl.ds(base, b_per_w)])

    return gather


def _make_scatter(n_acc, out_ch, ep):
    edges_w = ep // NS
    half = edges_w // 2
    n_half = half // CHUNK
    n_ch_w = edges_w // CHUNK
    rows_w = n_acc // NS
    mesh = plsc.VectorSubcoreMesh(core_axis_name="c", subcore_axis_name="s",
                                  num_cores=NC, num_subcores=NS)

    @functools.partial(
        pl.kernel, mesh=mesh,
        out_type=jax.ShapeDtypeStruct((n_acc, out_ch), jnp.float32),
        scratch_types=[
            pltpu.VMEM((half, out_ch), jnp.float32),
            pltpu.VMEM((n_ch_w, CHUNK), jnp.int32),
            pltpu.VMEM_SHARED((n_acc, out_ch), jnp.float32),
        ],
        compiler_params=pltpu.CompilerParams(use_tc_tiling_on_sc=False),
    )
    def scatter(oute_hbm, jidx_hbm, zeros_hbm, res_hbm, rows_v, jidx_v, res_sh):
        cid = lax.axis_index("c")
        sid = lax.axis_index("s")

        @pl.when(cid == 0)
        def _():
            pltpu.sync_copy(zeros_hbm.at[pl.ds(sid * rows_w, rows_w)],
                            res_sh.at[pl.ds(sid * rows_w, rows_w)])
            pltpu.sync_copy(jidx_hbm.at[sid], jidx_v)
            plsc.subcore_barrier()
            for h in range(2):
                pltpu.sync_copy(
                    oute_hbm.at[pl.ds(sid * edges_w + h * half, half)], rows_v)

                def body(j, carry):
                    pltpu.sync_copy(rows_v.at[pl.ds(j * CHUNK, CHUNK)],
                                    res_sh.at[jidx_v.at[h * n_half + j]],
                                    add=True)
                    return carry

                lax.fori_loop(0, n_half, body, 0)
            plsc.subcore_barrier()
            pltpu.sync_copy(res_sh.at[pl.ds(sid * rows_w, rows_w)],
                            res_hbm.at[pl.ds(sid * rows_w, rows_w)])

    return scatter


def _make_tc(e, ep, k_pts, in_ch, out_ch, block):
    kio = k_pts * in_ch

    def body(x_ref, f_ref, m_ref, w_ref, out_ref):
        pid = pl.program_id(0)
        x0 = x_ref[0:1, :]
        x1 = x_ref[1:2, :]
        x2 = x_ref[2:3, :]

        def d2(k):
            e0 = x0 - m_ref[k:k + 1, 0:1]
            e1 = x1 - m_ref[k:k + 1, 1:2]
            e2 = x2 - m_ref[k:k + 1, 2:3]
            return e0 * e0 + e1 * e1 + e2 * e2

        q = jnp.concatenate([d2(k) for k in range(k_pts)], axis=0)
        qmin = jnp.min(q, axis=0, keepdims=True)
        kio_t = lax.broadcasted_iota(jnp.int32, (k_pts, block), 0)
        selk_t = jnp.min(jnp.where(q == qmin, kio_t, k_pts),
                         axis=0, keepdims=True)
        selk = jnp.transpose(selk_t)

        kcol = lax.broadcasted_iota(jnp.int32, (block, kio), 1) // in_ch
        f = f_ref[...]
        fx = jnp.where(kcol == selk,
                       jnp.concatenate([f] * k_pts, axis=1),
                       jnp.float32(0.0))
        out = jnp.dot(fx, w_ref[...], preferred_element_type=jnp.float32)
        rid = pid * block + lax.broadcasted_iota(jnp.int32, (block, 1), 0)
        out_ref[...] = jnp.where(rid < e, out, jnp.float32(0.0))

    return pl.pallas_call(
        body,
        grid=(ep // block,),
        in_specs=[
            pl.BlockSpec((4, block), lambda i: (0, i)),
            pl.BlockSpec((block, in_ch), lambda i: (i, 0)),
            pl.BlockSpec((k_pts, 4), lambda i: (0, 0)),
            pl.BlockSpec((kio, out_ch), lambda i: (0, 0)),
        ],
        out_specs=pl.BlockSpec((block, out_ch), lambda i: (i, 0)),
        out_shape=jax.ShapeDtypeStruct((ep, out_ch), jnp.float32),
    )


def kernel(ijk, jkl, triangle_features, hood_coords, mu, W):
    e = ijk.shape[0]
    n_tri, in_ch = triangle_features.shape
    out_ch, k_pts, _ = W.shape

    align = NW * CHUNK
    ep = ((e + align - 1) // align) * align
    pad = ep - e

    ijk3d = jnp.pad(ijk, (0, pad)).reshape(NW, ep // (NW * CHUNK), CHUNK)
    jkl3d = jnp.pad(jkl, (0, pad)).reshape(NS, ep // (NS * CHUNK), CHUNK)
    xaug = jnp.concatenate(
        [jnp.pad(hood_coords, ((0, pad), (0, 0))).T,
         jnp.ones((1, ep), jnp.float32)], axis=0)
    msel = jnp.pad(mu[0], ((0, 0), (0, 1)))
    wflat = jnp.transpose(W, (1, 2, 0)).reshape(k_pts * in_ch, out_ch)

    n_acc = ((n_tri + 8 * NS - 1) // (8 * NS)) * (8 * NS)
    zeros = jnp.zeros((n_acc, out_ch), jnp.float32)
    tf_pad = jnp.pad(triangle_features, ((0, n_acc - n_tri), (0, 0)))

    f = _make_gather(n_tri, n_acc, in_ch, ep)(tf_pad, ijk3d)
    out_e = _make_tc(e, ep, k_pts, in_ch, out_ch, 2048)(xaug, f, msel, wflat)
    res = _make_scatter(n_acc, out_ch, ep)(out_e, jkl3d, zeros)
    return res[:n_tri]


# speedup vs baseline: 1.1472x; 1.1472x over previous
import functools

import jax
import jax.numpy as jnp
from jax import lax
from jax.experimental import pallas as pl
from jax.experimental.pallas import tpu as pltpu
from jax.experimental.pallas import tpu_sc as plsc

NC = 2
NS = 16
NW = NC * NS
CHUNK = 128


def _make_gather(n_tri, n_stage, in_ch, ep):
    b_per_w = ep // NW
    n_ch = b_per_w // CHUNK
    rows_w = n_stage // NS
    mesh = plsc.VectorSubcoreMesh(core_axis_name="c", subcore_axis_name="s",
                                  num_cores=NC, num_subcores=NS)

    @functools.partial(
        pl.kernel, mesh=mesh,
        out_type=jax.ShapeDtypeStruct((ep, in_ch), jnp.float32),
        scratch_types=[
            pltpu.VMEM((n_ch, CHUNK), jnp.int32),
            pltpu.VMEM((b_per_w, in_ch), jnp.float32),
            pltpu.VMEM_SHARED((n_stage, in_ch), jnp.float32),
            pltpu.SemaphoreType.DMA,
        ],
        compiler_params=pltpu.CompilerParams(use_tc_tiling_on_sc=False),
    )
    def gather(tf_hbm, idx_hbm, f_hbm, idx_v, rows_v, tf_sh, sem):
        cid = lax.axis_index("c")
        sid = lax.axis_index("s")
        wid = sid * NC + cid
        base = wid * b_per_w
        pltpu.sync_copy(tf_hbm.at[pl.ds(sid * rows_w, rows_w)],
                        tf_sh.at[pl.ds(sid * rows_w, rows_w)])
        pltpu.sync_copy(idx_hbm.at[wid], idx_v)
        plsc.subcore_barrier()

        def fire(j, carry):
            pltpu.async_copy(tf_sh.at[idx_v.at[j]],
                             rows_v.at[pl.ds(j * CHUNK, CHUNK)], sem)
            return carry

        lax.fori_loop(0, n_ch, fire, 0)
        pltpu.make_async_copy(tf_hbm.at[pl.ds(0, b_per_w)], rows_v, sem).wait()
        pltpu.sync_copy(rows_v, f_hbm.at[pl.ds(base, b_per_w)])

    return gather


def _make_scatter(n_acc, out_ch, ep):
    edges_w = ep // NS
    half = edges_w // 2
    n_half = half // CHUNK
    n_ch_w = edges_w // CHUNK
    rows_w = n_acc // NS
    mesh = plsc.VectorSubcoreMesh(core_axis_name="c", subcore_axis_name="s",
                                  num_cores=NC, num_subcores=NS)

    hch = out_ch // NC

    @functools.partial(
        pl.kernel, mesh=mesh,
        out_type=jax.ShapeDtypeStruct((n_acc, out_ch), jnp.float32),
        scratch_types=[
            pltpu.VMEM((half, hch), jnp.float32),
            pltpu.VMEM((n_ch_w, CHUNK), jnp.int32),
            pltpu.VMEM_SHARED((n_acc, hch), jnp.float32),
            pltpu.SemaphoreType.DMA,
        ],
        compiler_params=pltpu.CompilerParams(use_tc_tiling_on_sc=False),
    )
    def scatter(oute_hbm, jidx_hbm, zeros_hbm, res_hbm, rows_v, jidx_v,
                res_sh, sem):
        cid = lax.axis_index("c")
        sid = lax.axis_index("s")
        c0 = cid * hch
        pltpu.sync_copy(zeros_hbm.at[pl.ds(sid * rows_w, rows_w)],
                        res_sh.at[pl.ds(sid * rows_w, rows_w)])
        pltpu.sync_copy(jidx_hbm.at[sid], jidx_v)
        plsc.subcore_barrier()
        for h in range(2):
            pltpu.sync_copy(
                oute_hbm.at[pl.ds(sid * edges_w + h * half, half),
                            pl.ds(c0, hch)], rows_v)

            def body(j, carry):
                pltpu.async_copy(rows_v.at[pl.ds(j * CHUNK, CHUNK)],
                                 res_sh.at[jidx_v.at[h * n_half + j]],
                                 sem, add=True)
                return carry

            lax.fori_loop(0, n_half, body, 0)
            pltpu.make_async_copy(rows_v, res_sh.at[pl.ds(0, half)],
                                  sem).wait()
        plsc.subcore_barrier()
        pltpu.sync_copy(res_sh.at[pl.ds(sid * rows_w, rows_w)],
                        res_hbm.at[pl.ds(sid * rows_w, rows_w),
                                   pl.ds(c0, hch)])

    return scatter


def _make_tc(e, ep, k_pts, in_ch, out_ch, block):
    kio = k_pts * in_ch

    def body(x_ref, f_ref, m_ref, w_ref, out_ref):
        pid = pl.program_id(0)
        x0 = x_ref[0:1, :]
        x1 = x_ref[1:2, :]
        x2 = x_ref[2:3, :]

        def d2(k):
            e0 = x0 - m_ref[k:k + 1, 0:1]
            e1 = x1 - m_ref[k:k + 1, 1:2]
            e2 = x2 - m_ref[k:k + 1, 2:3]
            return e0 * e0 + e1 * e1 + e2 * e2

        q = jnp.concatenate([d2(k) for k in range(k_pts)], axis=0)
        qmin = jnp.min(q, axis=0, keepdims=True)
        kio_t = lax.broadcasted_iota(jnp.int32, (k_pts, block), 0)
        selk_t = jnp.min(jnp.where(q == qmin, kio_t, k_pts),
                         axis=0, keepdims=True)
        selk = jnp.transpose(selk_t)

        kcol = lax.broadcasted_iota(jnp.int32, (block, kio), 1) // in_ch
        f = f_ref[...]
        fx = jnp.where(kcol == selk,
                       jnp.concatenate([f] * k_pts, axis=1),
                       jnp.float32(0.0))
        out = jnp.dot(fx, w_ref[...], preferred_element_type=jnp.float32)
        rid = pid * block + lax.broadcasted_iota(jnp.int32, (block, 1), 0)
        out_ref[...] = jnp.where(rid < e, out, jnp.float32(0.0))

    return pl.pallas_call(
        body,
        grid=(ep // block,),
        in_specs=[
            pl.BlockSpec((4, block), lambda i: (0, i)),
            pl.BlockSpec((block, in_ch), lambda i: (i, 0)),
            pl.BlockSpec((k_pts, 4), lambda i: (0, 0)),
            pl.BlockSpec((kio, out_ch), lambda i: (0, 0)),
        ],
        out_specs=pl.BlockSpec((block, out_ch), lambda i: (i, 0)),
        out_shape=jax.ShapeDtypeStruct((ep, out_ch), jnp.float32),
    )


def kernel(ijk, jkl, triangle_features, hood_coords, mu, W):
    e = ijk.shape[0]
    n_tri, in_ch = triangle_features.shape
    out_ch, k_pts, _ = W.shape

    align = NW * CHUNK
    ep = ((e + align - 1) // align) * align
    pad = ep - e

    ijk3d = jnp.pad(ijk, (0, pad)).reshape(NW, ep // (NW * CHUNK), CHUNK)
    jkl3d = jnp.pad(jkl, (0, pad)).reshape(NS, ep // (NS * CHUNK), CHUNK)
    xaug = jnp.concatenate(
        [jnp.pad(hood_coords, ((0, pad), (0, 0))).T,
         jnp.ones((1, ep), jnp.float32)], axis=0)
    msel = jnp.pad(mu[0], ((0, 0), (0, 1)))
    wflat = jnp.transpose(W, (1, 2, 0)).reshape(k_pts * in_ch, out_ch)

    n_acc = ((n_tri + 8 * NS - 1) // (8 * NS)) * (8 * NS)
    zeros = jnp.zeros((n_acc, out_ch // NC), jnp.float32)
    tf_pad = jnp.pad(triangle_features, ((0, n_acc - n_tri), (0, 0)))

    f = _make_gather(n_tri, n_acc, in_ch, ep)(tf_pad, ijk3d)
    out_e = _make_tc(e, ep, k_pts, in_ch, out_ch, 2048)(xaug, f, msel, wflat)
    res = _make_scatter(n_acc, out_ch, ep)(out_e, jkl3d, zeros)
    return res[:n_tri]


# speedup vs baseline: 1.2009x; 1.0468x over previous
import functools

import jax
import jax.numpy as jnp
from jax import lax
from jax.experimental import pallas as pl
from jax.experimental.pallas import tpu as pltpu
from jax.experimental.pallas import tpu_sc as plsc

NC = 2
NS = 16
NW = NC * NS
CHUNK = 128


def _make_gather(n_tri, n_stage, in_ch, ep):
    b_per_w = ep // NW
    n_ch = b_per_w // CHUNK
    rows_w = n_stage // NS
    mesh = plsc.VectorSubcoreMesh(core_axis_name="c", subcore_axis_name="s",
                                  num_cores=NC, num_subcores=NS)

    @functools.partial(
        pl.kernel, mesh=mesh,
        out_type=jax.ShapeDtypeStruct((ep, in_ch), jnp.float32),
        scratch_types=[
            pltpu.VMEM((n_ch, CHUNK), jnp.int32),
            pltpu.VMEM((b_per_w, in_ch), jnp.float32),
            pltpu.VMEM_SHARED((n_stage, in_ch), jnp.float32),
            pltpu.SemaphoreType.DMA,
        ],
        compiler_params=pltpu.CompilerParams(use_tc_tiling_on_sc=False),
    )
    def gather(tf_hbm, idx_hbm, f_hbm, idx_v, rows_v, tf_sh, sem):
        cid = lax.axis_index("c")
        sid = lax.axis_index("s")
        wid = sid * NC + cid
        base = wid * b_per_w
        pltpu.sync_copy(tf_hbm.at[pl.ds(sid * rows_w, rows_w)],
                        tf_sh.at[pl.ds(sid * rows_w, rows_w)])
        pltpu.sync_copy(idx_hbm.at[wid], idx_v)
        plsc.subcore_barrier()

        def fire(j, carry):
            pltpu.async_copy(tf_sh.at[idx_v.at[j]],
                             rows_v.at[pl.ds(j * CHUNK, CHUNK)], sem)
            return carry

        lax.fori_loop(0, n_ch, fire, 0)
        pltpu.make_async_copy(tf_hbm.at[pl.ds(0, b_per_w)], rows_v, sem).wait()
        pltpu.sync_copy(rows_v, f_hbm.at[pl.ds(base, b_per_w)])

    return gather


def _make_scatter(n_acc, out_ch, ep):
    edges_w = ep // NS
    half = edges_w // 2
    n_half = half // CHUNK
    n_ch_w = edges_w // CHUNK
    rows_w = n_acc // NS
    mesh = plsc.VectorSubcoreMesh(core_axis_name="c", subcore_axis_name="s",
                                  num_cores=NC, num_subcores=NS)

    hch = out_ch // NC

    @functools.partial(
        pl.kernel, mesh=mesh,
        out_type=jax.ShapeDtypeStruct((n_acc, out_ch), jnp.float32),
        scratch_types=[
            pltpu.VMEM((half, hch), jnp.float32),
            pltpu.VMEM((n_ch_w, CHUNK), jnp.int32),
            pltpu.VMEM_SHARED((n_acc, hch), jnp.float32),
            pltpu.SemaphoreType.DMA,
        ],
        compiler_params=pltpu.CompilerParams(use_tc_tiling_on_sc=False),
    )
    def scatter(oute_hbm, jidx_hbm, zeros_hbm, res_hbm, rows_v, jidx_v,
                res_sh, sem):
        cid = lax.axis_index("c")
        sid = lax.axis_index("s")
        c0 = cid * hch
        pltpu.sync_copy(zeros_hbm.at[pl.ds(sid * rows_w, rows_w)],
                        res_sh.at[pl.ds(sid * rows_w, rows_w)])
        pltpu.sync_copy(jidx_hbm.at[sid], jidx_v)
        plsc.subcore_barrier()
        for h in range(2):
            pltpu.sync_copy(
                oute_hbm.at[pl.ds(sid * edges_w + h * half, half),
                            pl.ds(c0, hch)], rows_v)

            def body(j, carry):
                pltpu.async_copy(rows_v.at[pl.ds(j * CHUNK, CHUNK)],
                                 res_sh.at[jidx_v.at[h * n_half + j]],
                                 sem, add=True)
                return carry

            lax.fori_loop(0, n_half, body, 0)
            pltpu.make_async_copy(rows_v, res_sh.at[pl.ds(0, half)],
                                  sem).wait()
        plsc.subcore_barrier()
        pltpu.sync_copy(res_sh.at[pl.ds(sid * rows_w, rows_w)],
                        res_hbm.at[pl.ds(sid * rows_w, rows_w),
                                   pl.ds(c0, hch)])

    return scatter


def _make_tc(e, ep, k_pts, in_ch, out_ch, block):
    kio = k_pts * in_ch

    def body(x_ref, f_ref, m_ref, w_ref, out_ref):
        x0 = x_ref[0:1, :]
        x1 = x_ref[1:2, :]
        x2 = x_ref[2:3, :]

        def d2(k):
            e0 = x0 - m_ref[k:k + 1, 0:1]
            e1 = x1 - m_ref[k:k + 1, 1:2]
            e2 = x2 - m_ref[k:k + 1, 2:3]
            return e0 * e0 + e1 * e1 + e2 * e2

        q = jnp.concatenate([d2(k) for k in range(k_pts)], axis=0)
        qmin = jnp.min(q, axis=0, keepdims=True)
        kio_t = lax.broadcasted_iota(jnp.int32, (k_pts, block), 0)
        selk_t = jnp.min(jnp.where(q == qmin, kio_t, k_pts),
                         axis=0, keepdims=True)
        selk = jnp.transpose(selk_t)

        kcol = lax.broadcasted_iota(jnp.int32, (block, kio), 1) // in_ch
        f = f_ref[...]
        fx = jnp.where(kcol == selk,
                       jnp.concatenate([f] * k_pts, axis=1),
                       jnp.float32(0.0))
        out_ref[...] = jnp.dot(fx, w_ref[...],
                               preferred_element_type=jnp.float32)

    return pl.pallas_call(
        body,
        grid=(ep // block,),
        in_specs=[
            pl.BlockSpec((4, block), lambda i: (0, i)),
            pl.BlockSpec((block, in_ch), lambda i: (i, 0)),
            pl.BlockSpec((k_pts, 4), lambda i: (0, 0)),
            pl.BlockSpec((kio, out_ch), lambda i: (0, 0)),
        ],
        out_specs=pl.BlockSpec((block, out_ch), lambda i: (i, 0)),
        out_shape=jax.ShapeDtypeStruct((ep, out_ch), jnp.float32),
    )


def kernel(ijk, jkl, triangle_features, hood_coords, mu, W):
    e = ijk.shape[0]
    n_tri, in_ch = triangle_features.shape
    out_ch, k_pts, _ = W.shape

    align = NW * CHUNK
    ep = ((e + align - 1) // align) * align
    pad = ep - e

    ijk3d = jnp.pad(ijk, (0, pad)).reshape(NW, ep // (NW * CHUNK), CHUNK)
    jkl3d = jnp.pad(jkl, (0, pad), constant_values=n_tri).reshape(
        NS, ep // (NS * CHUNK), CHUNK)
    xaug = jnp.concatenate(
        [jnp.pad(hood_coords, ((0, pad), (0, 0))).T,
         jnp.ones((1, ep), jnp.float32)], axis=0)
    msel = jnp.pad(mu[0], ((0, 0), (0, 1)))
    wflat = jnp.transpose(W, (1, 2, 0)).reshape(k_pts * in_ch, out_ch)

    n_acc = ((n_tri + 8 * NS - 1) // (8 * NS)) * (8 * NS)
    zeros = jnp.zeros((n_acc, out_ch // NC), jnp.float32)
    tf_pad = jnp.pad(triangle_features, ((0, n_acc - n_tri), (0, 0)))

    f = _make_gather(n_tri, n_acc, in_ch, ep)(tf_pad, ijk3d)
    out_e = _make_tc(e, ep, k_pts, in_ch, out_ch, 4096)(xaug, f, msel, wflat)
    res = _make_scatter(n_acc, out_ch, ep)(out_e, jkl3d, zeros)
    return res[:n_tri]


# speedup vs baseline: 1.3003x; 1.0827x over previous
import functools

import jax
import jax.numpy as jnp
from jax import lax
from jax.experimental import pallas as pl
from jax.experimental.pallas import tpu as pltpu
from jax.experimental.pallas import tpu_sc as plsc

NC = 2
NS = 16
NW = NC * NS
CHUNK = 128


def _make_gather(n_tri, n_stage, in_ch, ep):
    b_per_w = ep // NW
    n_ch = b_per_w // CHUNK
    rows_w = n_stage // NS
    mesh = plsc.VectorSubcoreMesh(core_axis_name="c", subcore_axis_name="s",
                                  num_cores=NC, num_subcores=NS)

    @functools.partial(
        pl.kernel, mesh=mesh,
        out_type=jax.ShapeDtypeStruct((ep, in_ch), jnp.float32),
        scratch_types=[
            pltpu.VMEM((n_ch, CHUNK), jnp.int32),
            pltpu.VMEM((b_per_w, in_ch), jnp.float32),
            pltpu.VMEM_SHARED((n_stage, in_ch), jnp.float32),
            pltpu.SemaphoreType.DMA,
        ],
        compiler_params=pltpu.CompilerParams(use_tc_tiling_on_sc=False),
    )
    def gather(tf_hbm, idx_hbm, f_hbm, idx_v, rows_v, tf_sh, sem):
        cid = lax.axis_index("c")
        sid = lax.axis_index("s")
        wid = sid * NC + cid
        base = wid * b_per_w
        pltpu.sync_copy(tf_hbm.at[pl.ds(sid * rows_w, rows_w)],
                        tf_sh.at[pl.ds(sid * rows_w, rows_w)])
        pltpu.sync_copy(idx_hbm.at[wid], idx_v)
        plsc.subcore_barrier()

        def fire(j, carry):
            pltpu.async_copy(tf_sh.at[idx_v.at[j]],
                             rows_v.at[pl.ds(j * CHUNK, CHUNK)], sem)
            return carry

        lax.fori_loop(0, n_ch, fire, 0)
        pltpu.make_async_copy(tf_hbm.at[pl.ds(0, b_per_w)], rows_v, sem).wait()
        pltpu.sync_copy(rows_v, f_hbm.at[pl.ds(base, b_per_w)])

    return gather


def _make_scatter(n_acc, out_ch, ep):
    edges_w = ep // NS
    half = edges_w // 2
    n_half = half // CHUNK
    n_ch_w = edges_w // CHUNK
    rows_w = n_acc // NS
    mesh = plsc.VectorSubcoreMesh(core_axis_name="c", subcore_axis_name="s",
                                  num_cores=NC, num_subcores=NS)

    hch = out_ch // NC

    @functools.partial(
        pl.kernel, mesh=mesh,
        out_type=jax.ShapeDtypeStruct((n_acc, out_ch), jnp.float32),
        scratch_types=[
            pltpu.VMEM((half, hch), jnp.float32),
            pltpu.VMEM((n_ch_w, CHUNK), jnp.int32),
            pltpu.VMEM_SHARED((n_acc, hch), jnp.float32),
            pltpu.SemaphoreType.DMA,
        ],
        compiler_params=pltpu.CompilerParams(use_tc_tiling_on_sc=False),
    )
    def scatter(oute_hbm, jidx_hbm, zeros_hbm, res_hbm, rows_v, jidx_v,
                res_sh, sem):
        cid = lax.axis_index("c")
        sid = lax.axis_index("s")
        c0 = cid * hch
        pltpu.sync_copy(zeros_hbm.at[pl.ds(sid * rows_w, rows_w)],
                        res_sh.at[pl.ds(sid * rows_w, rows_w)])
        pltpu.sync_copy(jidx_hbm.at[sid], jidx_v)
        plsc.subcore_barrier()
        for h in range(2):
            pltpu.sync_copy(
                oute_hbm.at[pl.ds(sid * edges_w + h * half, half),
                            pl.ds(c0, hch)], rows_v)

            def body(j, carry):
                pltpu.async_copy(rows_v.at[pl.ds(j * CHUNK, CHUNK)],
                                 res_sh.at[jidx_v.at[h * n_half + j]],
                                 sem, add=True)
                return carry

            lax.fori_loop(0, n_half, body, 0)
            pltpu.make_async_copy(rows_v, res_sh.at[pl.ds(0, half)],
                                  sem).wait()
        plsc.subcore_barrier()
        pltpu.sync_copy(res_sh.at[pl.ds(sid * rows_w, rows_w)],
                        res_hbm.at[pl.ds(sid * rows_w, rows_w),
                                   pl.ds(c0, hch)])

    return scatter


def _make_tc(e, ep, k_pts, in_ch, out_ch, block):
    kio = k_pts * in_ch

    def body(x_ref, f_ref, m_ref, w_ref, out_ref):
        x0 = x_ref[0:1, :]
        x1 = x_ref[1:2, :]
        x2 = x_ref[2:3, :]

        def d2(k):
            e0 = x0 - m_ref[k:k + 1, 0:1]
            e1 = x1 - m_ref[k:k + 1, 1:2]
            e2 = x2 - m_ref[k:k + 1, 2:3]
            return e0 * e0 + e1 * e1 + e2 * e2

        q = jnp.concatenate([d2(k) for k in range(k_pts)], axis=0)
        qmin = jnp.min(q, axis=0, keepdims=True)
        kio_t = lax.broadcasted_iota(jnp.int32, (k_pts, block), 0)
        selk_t = jnp.min(jnp.where(q == qmin, kio_t, k_pts),
                         axis=0, keepdims=True)
        selk = jnp.transpose(selk_t)

        kcol = lax.broadcasted_iota(jnp.int32, (block, kio), 1) // in_ch
        f = f_ref[...].astype(jnp.bfloat16)
        fx = jnp.where(kcol == selk,
                       jnp.concatenate([f] * k_pts, axis=1),
                       jnp.bfloat16(0.0))
        out_ref[...] = jnp.dot(fx, w_ref[...],
                               preferred_element_type=jnp.float32)

    return pl.pallas_call(
        body,
        grid=(ep // block,),
        in_specs=[
            pl.BlockSpec((4, block), lambda i: (0, i)),
            pl.BlockSpec((block, in_ch), lambda i: (i, 0)),
            pl.BlockSpec((k_pts, 4), lambda i: (0, 0)),
            pl.BlockSpec((kio, out_ch), lambda i: (0, 0)),
        ],
        out_specs=pl.BlockSpec((block, out_ch), lambda i: (i, 0)),
        out_shape=jax.ShapeDtypeStruct((ep, out_ch), jnp.float32),
    )


def kernel(ijk, jkl, triangle_features, hood_coords, mu, W):
    e = ijk.shape[0]
    n_tri, in_ch = triangle_features.shape
    out_ch, k_pts, _ = W.shape

    align = NW * CHUNK
    ep = ((e + align - 1) // align) * align
    pad = ep - e

    ijk3d = jnp.pad(ijk, (0, pad)).reshape(NW, ep // (NW * CHUNK), CHUNK)
    jkl3d = jnp.pad(jkl, (0, pad), constant_values=n_tri).reshape(
        NS, ep // (NS * CHUNK), CHUNK)
    xaug = jnp.concatenate(
        [jnp.pad(hood_coords, ((0, pad), (0, 0))).T,
         jnp.ones((1, ep), jnp.float32)], axis=0)
    msel = jnp.pad(mu[0], ((0, 0), (0, 1)))
    wflat = jnp.transpose(W, (1, 2, 0)).reshape(
        k_pts * in_ch, out_ch).astype(jnp.bfloat16)

    n_acc = ((n_tri + 8 * NS - 1) // (8 * NS)) * (8 * NS)
    zeros = jnp.zeros((n_acc, out_ch // NC), jnp.float32)
    tf_pad = jnp.pad(triangle_features, ((0, n_acc - n_tri), (0, 0)))

    f = _make_gather(n_tri, n_acc, in_ch, ep)(tf_pad, ijk3d)
    out_e = _make_tc(e, ep, k_pts, in_ch, out_ch, 4096)(xaug, f, msel, wflat)
    res = _make_scatter(n_acc, out_ch, ep)(out_e, jkl3d, zeros)
    return res[:n_tri]
